# baseline (speedup 1.0000x reference)
import functools

import jax
import jax.numpy as jnp
from jax import lax
from jax.experimental import pallas as pl
from jax.experimental.pallas import tpu as pltpu
from jax.experimental.pallas import tpu_sc as plsc

_VOCAB = 1000000
_EMSIZE = 64
_B = 4096
_L = 50

_NC = 2
_NS = 16
_NW = _NC * _NS
_RPW = _B // _NW
_NR = 1
_NCH = _RPW // _NR
_NBUF = 8
_NROUND = _NCH // _NBUF

_mesh = plsc.VectorSubcoreMesh(core_axis_name="c", subcore_axis_name="s")


@functools.partial(
    pl.kernel,
    mesh=_mesh,
    compiler_params=pltpu.CompilerParams(use_tc_tiling_on_sc=False),
    out_type=jax.ShapeDtypeStruct((_B, _L, _EMSIZE), jnp.float32),
    scratch_types=(
        [pltpu.VMEM((_RPW, _L), jnp.int32),
         pltpu.VMEM((_NBUF, _L, _EMSIZE), jnp.float32)]
        + [pltpu.SemaphoreType.DMA] * (2 * _NBUF)
    ),
)
def _embed_sc(idx_hbm, table_hbm, out_hbm, idx_v, buf_v, *sems):
    gs, ws = sems[:_NBUF], sems[_NBUF:]
    wid = lax.axis_index("s") * _NC + lax.axis_index("c")
    base = wid * _RPW
    pltpu.sync_copy(idx_hbm.at[pl.ds(base, _RPW)], idx_v)

    def start_gather(b, j):
        pltpu.async_copy(table_hbm.at[idx_v.at[j]], buf_v.at[b], gs[b])

    def wait_gather(b, j):
        pltpu.make_async_copy(
            table_hbm.at[idx_v.at[j]], buf_v.at[b], gs[b]).wait()

    def start_wb(b, j):
        pltpu.async_copy(buf_v.at[b], out_hbm.at[base + j], ws[b])

    def wait_wb(b, j):
        pltpu.make_async_copy(
            buf_v.at[b], out_hbm.at[base + j], ws[b]).wait()

    for b in range(_NBUF):
        start_gather(b, b)

    def round_body(g, carry):
        for b in range(_NBUF):
            j = g * _NBUF + b
            wait_gather(b, j)
            start_wb(b, j)
        for b in range(_NBUF):
            j = g * _NBUF + b
            wait_wb(b, j)
            start_gather(b, j + _NBUF)
        return carry

    lax.fori_loop(0, _NROUND - 1, round_body, 0)

    gl = _NROUND - 1
    for b in range(_NBUF):
        j = gl * _NBUF + b
        wait_gather(b, j)
        start_wb(b, j)
    for b in range(_NBUF):
        wait_wb(b, gl * _NBUF + b)


def kernel(input_variable, embedding_weight):
    idx = input_variable
    if idx.dtype != jnp.int32:
        idx = idx.astype(jnp.int32)
    return _embed_sc(idx, embedding_weight)


# speedup vs baseline: 1.0387x; 1.0387x over previous
import functools

import jax
import jax.numpy as jnp
from jax import lax
from jax.experimental import pallas as pl
from jax.experimental.pallas import tpu as pltpu
from jax.experimental.pallas import tpu_sc as plsc

_VOCAB = 1000000
_EMSIZE = 64
_PADE = 128
_B = 4096
_L = 50

_NC = 2
_NS = 16
_NW = _NC * _NS
_RPW = _B // _NW
_NBUF = 8
_NROUND = _RPW // _NBUF

_mesh = plsc.VectorSubcoreMesh(core_axis_name="c", subcore_axis_name="s")


@functools.partial(
    pl.kernel,
    mesh=_mesh,
    compiler_params=pltpu.CompilerParams(use_tc_tiling_on_sc=False),
    out_type=jax.ShapeDtypeStruct((_B, _L, _EMSIZE), jnp.float32),
    scratch_types=(
        [pltpu.VMEM((_RPW, _L), jnp.int32),
         pltpu.VMEM((_NBUF, _L, _PADE), jnp.float32)]
        + [pltpu.SemaphoreType.DMA] * (2 * _NBUF)
    ),
)
def _embed_sc(idx_hbm, table_hbm, out_hbm, idx_v, buf_v, *sems):
    gs, ws = sems[:_NBUF], sems[_NBUF:]
    wid = lax.axis_index("s") * _NC + lax.axis_index("c")
    base = wid * _RPW
    pltpu.sync_copy(idx_hbm.at[pl.ds(base, _RPW)], idx_v)

    def start_gather(b, j):
        pltpu.async_copy(table_hbm.at[idx_v.at[j]], buf_v.at[b], gs[b])

    def wait_gather(b, j):
        pltpu.make_async_copy(
            table_hbm.at[idx_v.at[j]], buf_v.at[b], gs[b]).wait()

    def start_wb(b, j):
        pltpu.async_copy(
            buf_v.at[b, :, pl.ds(0, _EMSIZE)], out_hbm.at[base + j], ws[b])

    def wait_wb(b, j):
        pltpu.make_async_copy(
            buf_v.at[b, :, pl.ds(0, _EMSIZE)], out_hbm.at[base + j],
            ws[b]).wait()

    for b in range(_NBUF):
        start_gather(b, b)

    def round_body(g, carry):
        for b in range(_NBUF):
            j = g * _NBUF + b
            wait_gather(b, j)
            start_wb(b, j)
        for b in range(_NBUF):
            j = g * _NBUF + b
            wait_wb(b, j)
            start_gather(b, j + _NBUF)
        return carry

    lax.fori_loop(0, _NROUND - 1, round_body, 0)

    gl = _NROUND - 1
    for b in range(_NBUF):
        j = gl * _NBUF + b
        wait_gather(b, j)
        start_wb(b, j)
    for b in range(_NBUF):
        wait_wb(b, gl * _NBUF + b)


def kernel(input_variable, embedding_weight):
    idx = input_variable
    if idx.dtype != jnp.int32:
        idx = idx.astype(jnp.int32)
    table128 = jnp.pad(embedding_weight, ((0, 0), (0, _PADE - _EMSIZE)))
    return _embed_sc(idx, table128)


# speedup vs baseline: 1.3729x; 1.3218x over previous
import functools

import jax
import jax.numpy as jnp
from jax import lax
from jax.experimental import pallas as pl
from jax.experimental.pallas import tpu as pltpu
from jax.experimental.pallas import tpu_sc as plsc

_VOCAB = 1000000
_EMSIZE = 64
_PADE = 128
_B = 4096
_L = 50

_NC = 2
_NS = 16
_NW = _NC * _NS
_RPW = _B // _NW
_NBUF = 8
_NROUND = _RPW // _NBUF

_mesh = plsc.VectorSubcoreMesh(core_axis_name="c", subcore_axis_name="s")


_TBS = 4096
_TGRID = -(-_VOCAB // _TBS)


def _tpose_body(t_ref, o_ref):
    eye = jnp.eye(_EMSIZE, dtype=jnp.float32)
    t = jax.lax.dot_general(
        t_ref[...], eye, (((0,), (0,)), ((), ())),
        preferred_element_type=jnp.float32)
    o_ref[...] = jnp.concatenate(
        [t, jnp.zeros((_TBS, _PADE - _EMSIZE), jnp.float32)], axis=1)


_transpose_pad = pl.pallas_call(
    _tpose_body,
    grid=(_TGRID,),
    in_specs=[pl.BlockSpec((_EMSIZE, _TBS), lambda k: (0, k))],
    out_specs=pl.BlockSpec((_TBS, _PADE), lambda k: (k, 0)),
    out_shape=jax.ShapeDtypeStruct((_VOCAB, _PADE), jnp.float32),
)


@functools.partial(
    pl.kernel,
    mesh=_mesh,
    compiler_params=pltpu.CompilerParams(use_tc_tiling_on_sc=False),
    out_type=jax.ShapeDtypeStruct((_B, _L, _EMSIZE), jnp.float32),
    scratch_types=(
        [pltpu.VMEM((_RPW, _L), jnp.int32),
         pltpu.VMEM((_NBUF, _L, _PADE), jnp.float32)]
        + [pltpu.SemaphoreType.DMA] * (2 * _NBUF)
    ),
)
def _embed_sc(idx_hbm, table_hbm, out_hbm, idx_v, buf_v, *sems):
    gs, ws = sems[:_NBUF], sems[_NBUF:]
    wid = lax.axis_index("s") * _NC + lax.axis_index("c")
    base = wid * _RPW
    pltpu.sync_copy(idx_hbm.at[pl.ds(base, _RPW)], idx_v)

    def start_gather(b, j):
        pltpu.async_copy(table_hbm.at[idx_v.at[j]], buf_v.at[b], gs[b])

    def wait_gather(b, j):
        pltpu.make_async_copy(
            table_hbm.at[idx_v.at[j]], buf_v.at[b], gs[b]).wait()

    def start_wb(b, j):
        pltpu.async_copy(
            buf_v.at[b, :, pl.ds(0, _EMSIZE)], out_hbm.at[base + j], ws[b])

    def wait_wb(b, j):
        pltpu.make_async_copy(
            buf_v.at[b, :, pl.ds(0, _EMSIZE)], out_hbm.at[base + j],
            ws[b]).wait()

    for b in range(_NBUF):
        start_gather(b, b)

    def round_body(g, carry):
        for b in range(_NBUF):
            j = g * _NBUF + b
            wait_gather(b, j)
            start_wb(b, j)
        for b in range(_NBUF):
            j = g * _NBUF + b
            wait_wb(b, j)
            start_gather(b, j + _NBUF)
        return carry

    lax.fori_loop(0, _NROUND - 1, round_body, 0)

    gl = _NROUND - 1
    for b in range(_NBUF):
        j = gl * _NBUF + b
        wait_gather(b, j)
        start_wb(b, j)
    for b in range(_NBUF):
        wait_wb(b, gl * _NBUF + b)


def kernel(input_variable, embedding_weight):
    idx = input_variable
    if idx.dtype != jnp.int32:
        idx = idx.astype(jnp.int32)
    table128 = _transpose_pad(embedding_weight.T)
    return _embed_sc(idx, table128)
